# baseline (device time: 19931 ns/iter reference)
import jax
import jax.numpy as jnp
from jax import lax
from jax.experimental import pallas as pl
from jax.experimental.pallas import tpu as pltpu

K = 16


def _topk_quad4(a, b, c, d, k):
    h1, l1 = jnp.maximum(a, b), jnp.minimum(a, b)
    h2, l2 = jnp.maximum(c, d), jnp.minimum(c, d)
    s1, t = jnp.maximum(h1, h2), jnp.minimum(h1, h2)
    s4, u = jnp.minimum(l1, l2), jnp.maximum(l1, l2)
    s2, s3 = jnp.maximum(t, u), jnp.minimum(t, u)
    cols = []
    for i in range(k):
        m = jnp.max(s1, axis=1, keepdims=True)
        cols.append(m)
        if i < k - 1:
            adv = s1 == m
            s1 = jnp.where(adv, s2, s1)
            s2 = jnp.where(adv, s3, s2)
            s3 = jnp.where(adv, s4, s3)
            s4 = jnp.where(adv, -jnp.inf, s4)
    return jnp.concatenate(cols, axis=1)


def _topk_desc(v, k):
    q = v.shape[1] // 4
    return _topk_quad4(
        v[:, :q], v[:, q : 2 * q], v[:, 2 * q : 3 * q], v[:, 3 * q :], k
    )


def _topk_merge(a, b, k):
    h = a.shape[1] // 2
    return _topk_quad4(a[:, :h], a[:, h:], b[:, :h], b[:, h:], k)


def kernel(x):
    m, n_loc = x.shape
    m_half = m // 2

    def body(x_hbm, o_ref, xv_ref, a_ref, recv_ref, dma_sem, send_sems, recv_sems):
        my_x = lax.axis_index("x")
        my_y = lax.axis_index("y")
        peers = [
            (my_x, 1 - my_y),
            (1 - my_x, my_y),
            (1 - my_x, 1 - my_y),
        ]

        barrier = pltpu.get_barrier_semaphore()
        for p in peers:
            pl.semaphore_signal(
                barrier, inc=1, device_id=p,
                device_id_type=pl.DeviceIdType.MESH,
            )

        dma = pltpu.make_async_copy(
            x_hbm.at[pl.ds(my_y * m_half, m_half), :], xv_ref, dma_sem
        )
        dma.start()
        dma.wait()

        a_ref[:, :] = _topk_desc(xv_ref[:, :], K)

        pl.semaphore_wait(barrier, 3)

        rdmas = [None, None, None]
        for i in (2, 1, 0):
            r = pltpu.make_async_remote_copy(
                src_ref=a_ref,
                dst_ref=recv_ref.at[i],
                send_sem=send_sems.at[i],
                recv_sem=recv_sems.at[i],
                device_id=peers[i],
                device_id_type=pl.DeviceIdType.MESH,
            )
            r.start()
            rdmas[i] = r

        rdmas[1].wait_recv()
        o_ref[pl.ds(my_y * m_half, m_half), :] = _topk_merge(
            a_ref[:, :], recv_ref[1], K
        )

        rdmas[0].wait_recv()
        rdmas[2].wait_recv()
        o_ref[pl.ds((1 - my_y) * m_half, m_half), :] = _topk_merge(
            recv_ref[0], recv_ref[2], K
        )

        for r in rdmas:
            r.wait_send()

    return pl.pallas_call(
        body,
        out_shape=jax.ShapeDtypeStruct((m, K), jnp.float32),
        in_specs=[pl.BlockSpec(memory_space=pltpu.MemorySpace.HBM)],
        out_specs=pl.BlockSpec(memory_space=pltpu.VMEM),
        scratch_shapes=[
            pltpu.VMEM((m_half, n_loc), jnp.float32),
            pltpu.VMEM((m_half, K), jnp.float32),
            pltpu.VMEM((3, m_half, K), jnp.float32),
            pltpu.SemaphoreType.DMA,
            pltpu.SemaphoreType.DMA((3,)),
            pltpu.SemaphoreType.DMA((3,)),
        ],
        compiler_params=pltpu.CompilerParams(collective_id=0),
    )(x)


# device time: 19639 ns/iter; 1.0149x vs baseline; 1.0149x over previous
import jax
import jax.numpy as jnp
from jax import lax
from jax.experimental import pallas as pl
from jax.experimental.pallas import tpu as pltpu

K = 16


def _topk_desc(v, k):
    n = v.shape[1]
    return _topk_merge(v[:, : n // 2], v[:, n // 2 :], k)


def _topk_merge(va, vb, k):
    hi = jnp.maximum(va, vb)
    lo = jnp.minimum(va, vb)
    cols = []
    m = None
    for _ in range(k // 2):
        if m is None:
            whi, wlo = hi, lo
        else:
            whi = jnp.where(hi < m, hi, -jnp.inf)
            wlo = jnp.where(lo < m, lo, -jnp.inf)
        cur = jnp.maximum(whi, wlo)
        sec = jnp.minimum(whi, wlo)
        m1 = jnp.max(cur, axis=1, keepdims=True)
        cand = jnp.where(cur == m1, sec, cur)
        m2 = jnp.max(cand, axis=1, keepdims=True)
        cols.append(m1)
        cols.append(m2)
        m = m2
    return jnp.concatenate(cols, axis=1)


def kernel(x):
    m, n_loc = x.shape
    m_half = m // 2

    def body(x_hbm, o_ref, xv_ref, a_ref, recv_ref, dma_sem, send_sems, recv_sems):
        my_x = lax.axis_index("x")
        my_y = lax.axis_index("y")
        peers = [
            (my_x, 1 - my_y),
            (1 - my_x, my_y),
            (1 - my_x, 1 - my_y),
        ]

        barrier = pltpu.get_barrier_semaphore()
        for p in peers:
            pl.semaphore_signal(
                barrier, inc=1, device_id=p,
                device_id_type=pl.DeviceIdType.MESH,
            )

        dma = pltpu.make_async_copy(
            x_hbm.at[pl.ds(my_y * m_half, m_half), :], xv_ref, dma_sem
        )
        dma.start()
        dma.wait()

        a_ref[:, :] = _topk_desc(xv_ref[:, :], K)

        pl.semaphore_wait(barrier, 3)

        rdmas = [None, None, None]
        for i in (2, 1, 0):
            r = pltpu.make_async_remote_copy(
                src_ref=a_ref,
                dst_ref=recv_ref.at[i],
                send_sem=send_sems.at[i],
                recv_sem=recv_sems.at[i],
                device_id=peers[i],
                device_id_type=pl.DeviceIdType.MESH,
            )
            r.start()
            rdmas[i] = r

        rdmas[1].wait_recv()
        o_ref[pl.ds(my_y * m_half, m_half), :] = _topk_merge(
            a_ref[:, :], recv_ref[1], K
        )

        rdmas[0].wait_recv()
        rdmas[2].wait_recv()
        o_ref[pl.ds((1 - my_y) * m_half, m_half), :] = _topk_merge(
            recv_ref[0], recv_ref[2], K
        )

        for r in rdmas:
            r.wait_send()

    return pl.pallas_call(
        body,
        out_shape=jax.ShapeDtypeStruct((m, K), jnp.float32),
        in_specs=[pl.BlockSpec(memory_space=pltpu.MemorySpace.HBM)],
        out_specs=pl.BlockSpec(memory_space=pltpu.VMEM),
        scratch_shapes=[
            pltpu.VMEM((m_half, n_loc), jnp.float32),
            pltpu.VMEM((m_half, K), jnp.float32),
            pltpu.VMEM((3, m_half, K), jnp.float32),
            pltpu.SemaphoreType.DMA,
            pltpu.SemaphoreType.DMA((3,)),
            pltpu.SemaphoreType.DMA((3,)),
        ],
        compiler_params=pltpu.CompilerParams(collective_id=0),
    )(x)
